# Initial kernel scaffold; baseline (speedup 1.0000x reference)
import jax
import jax.numpy as jnp
from jax.experimental import pallas as pl


def kernel(rdr_cube, batch_size):
    raise NotImplementedError("write your pallas kernel here")


# speedup vs baseline: 18.2572x; 18.2572x over previous
import jax
import jax.numpy as jnp
from jax import lax
from jax.experimental import pallas as pl
from jax.experimental.pallas import tpu as pltpu
from jax.experimental.pallas import tpu_sc as plsc

B = 4
ZD, YD, XD = 32, 256, 256
N = ZD * YD * XD
NT = 16
SH = N // NT
CH = 2048
NCH = SH // CH
K0 = 1468005
K_KEEP = N - K0 - 1
R_OUT = B * K_KEEP
DUMP = R_OUT
OUT_PAD = R_OUT + 128
OW = 16
NBINS = 65536
CHB = NBINS // NT
CHR = CHB // 16
HW = 0.625
INT_MIN = -2147483648

_i32 = jnp.int32
_f32 = jnp.float32


def _bits(v):
  import struct
  return int.from_bytes(struct.pack('<f', v), 'little', signed=True)


B_Y = _bits(-40.0)
B_Z = _bits(-2.0)


def _monotonic(x):
  xi = lax.bitcast_convert_type(x, _i32)
  return jnp.where(xi < 0, jnp.bitwise_not(xi),
                   jnp.bitwise_or(xi, _i32(INT_MIN)))


def _scalar(vec):
  return jnp.max(vec)


def _kernel_body(rdr, out, buf, hist, acc, zb, aos, idx2d, padbuf,
                 totv, resv, minv, cntv, stage, sphist, sptot, spres,
                 spmin, spcnt, sem):
  c = lax.axis_index("c")
  t = lax.axis_index("s")
  iota = lax.iota(_i32, 16)

  def zero_rows(ref, nrows):
    def zh(i, _):
      ref[i] = jnp.zeros((16,), _i32)
      return 0
    lax.fori_loop(0, nrows, zh, 0)

  def put_row(sp, row, scalar_val):
    stage[...] = jnp.broadcast_to(scalar_val.astype(_i32), (16,))
    pltpu.sync_copy(stage, sp.at[row])

  def merge_hist_to_spmem():
    pltpu.sync_copy(zb, sphist.at[pl.ds(t * CHR, CHR)])
    plsc.subcore_barrier()
    def mg(j, _):
      for o in range(8):
        idx2d[j, pl.ds(o * 16, 16)] = j * 128 + o * 16 + iota
      pltpu.sync_copy(hist.at[pl.ds(j * 128, 128)],
                      sphist.at[idx2d.at[j]], add=True)
      return 0
    lax.fori_loop(0, NBINS // 16 // 128, mg, 0)
    plsc.subcore_barrier()

  def reduce_and_cumsum():
    pltpu.sync_copy(sphist.at[pl.ds(t * CHR, CHR)], acc)
    def csum(i, run):
      v = acc[i]
      acc[i] = plsc.cumsum(v) + run
      return run + jnp.sum(v)
    total = lax.fori_loop(0, CHR, csum, _i32(0))
    put_row(sptot, t, total)
    plsc.subcore_barrier()
    pltpu.sync_copy(sptot, totv)
    base = _i32(0)
    for r in range(NT):
      tr = _scalar(totv[r])
      base = base + jnp.where(r < t, tr, 0).astype(_i32)
    return base, total

  def rank_search(base, total, k, row_bin, row_cnt):
    def lp(i, cnt):
      v = acc[i] + base
      return cnt + jnp.sum(jnp.where(v <= k, 1, 0).astype(_i32))
    L = lax.fori_loop(0, CHR, lp, _i32(0))
    claim = jnp.logical_and(k >= base, k < base + total)

    @pl.when(claim)
    def _():
      lm = jnp.maximum(L - 1, 0)
      gv = plsc.load_gather(
          acc, [jnp.broadcast_to(lax.shift_right_logical(lm, 4), (16,)),
                jnp.broadcast_to(jnp.bitwise_and(lm, 15), (16,))])
      below = base + jnp.where(L > 0, _scalar(gv), 0).astype(_i32)
      put_row(spres, row_bin, t * CHB + L)
      if row_cnt is not None:
        put_row(spres, row_cnt, below)

  zero_rows(zb, CHR)

  def sample_body(sl, _):
    s = c * 2 + sl
    shard = t * SH

    zero_rows(hist, NBINS // 16)
    def p1_chunk(i, __):
      pltpu.sync_copy(rdr.at[s, pl.ds(shard + i * CH, CH)], buf)
      def vl(v, ___):
        x = buf[pl.ds(v * 16, 16)]
        hi = lax.shift_right_logical(_monotonic(x), 16)
        cnt, last = plsc.scan_count(hi)
        plsc.addupdate_scatter(
            hist, [lax.shift_right_logical(hi, 4), jnp.bitwise_and(hi, 15)],
            cnt, mask=last)
        return 0
      lax.fori_loop(0, CH // 16, vl, 0)
      return 0
    lax.fori_loop(0, NCH, p1_chunk, 0)
    merge_hist_to_spmem()

    base, total = reduce_and_cumsum()
    rank_search(base, total, _i32(K0), 0, 1)
    rank_search(base, total, _i32(K0 + 1), 2, 3)
    plsc.subcore_barrier()
    pltpu.sync_copy(spres, resv)
    b0 = _scalar(resv[0])
    cb0 = _scalar(resv[1])
    b1 = _scalar(resv[2])
    cb1 = _scalar(resv[3])

    zero_rows(hist, NBINS // 16)
    def p2_chunk(i, mlow):
      pltpu.sync_copy(rdr.at[s, pl.ds(shard + i * CH, CH)], buf)
      def vl(v, ml):
        x = buf[pl.ds(v * 16, 16)]
        m = _monotonic(x)
        hi = lax.shift_right_logical(m, 16)
        lo = jnp.bitwise_and(m, 65535)
        cnt, last = plsc.scan_count(lo, mask=hi == b0)
        plsc.addupdate_scatter(
            hist, [lax.shift_right_logical(lo, 4), jnp.bitwise_and(lo, 15)],
            cnt, mask=last)
        lv = jnp.where(hi == b1, lo, 70000)
        return jnp.minimum(ml, jnp.min(lv)).astype(_i32)
      return lax.fori_loop(0, CH // 16, vl, mlow)
    minlow = lax.fori_loop(0, NCH, p2_chunk, _i32(70000))
    put_row(spmin, t, minlow)
    merge_hist_to_spmem()

    base2, total2 = reduce_and_cumsum()
    rank_search(base2, total2, K0 - cb0, 4, None)
    rank_search(base2, total2, K0 + 1 - cb1, 5, None)
    plsc.subcore_barrier()
    pltpu.sync_copy(spres, resv)
    pltpu.sync_copy(spmin, minv)
    l0 = _scalar(resv[4])
    l1 = _scalar(resv[5])
    mlg = _i32(70000)
    for r in range(NT):
      mlg = jnp.minimum(mlg, _scalar(minv[r]))
    l1s = jnp.where(b1 == b0, l1, mlg).astype(_i32)
    v0m = jnp.bitwise_or(lax.shift_left(b0, 16), l0)
    v1m = jnp.bitwise_or(lax.shift_left(b1, 16), l1s)

    def m2f(mv):
      vec = jnp.broadcast_to(mv, (16,))
      bits = jnp.where(vec < 0, jnp.bitwise_and(vec, 2147483647),
                       jnp.bitwise_not(vec))
      return lax.bitcast_convert_type(bits.astype(_i32), _f32)
    v0f = m2f(v0m)
    v1f = m2f(v1m)
    threshv = v0f + _f32(HW) * (v1f - v0f)

    def cnt_chunk(i, cc):
      pltpu.sync_copy(rdr.at[s, pl.ds(shard + i * CH, CH)], buf)
      def vl(v, c2):
        x = buf[pl.ds(v * 16, 16)]
        return c2 + jnp.sum(jnp.where(x > threshv, 1, 0).astype(_i32))
      return lax.fori_loop(0, CH // 16, vl, cc)
    mycnt = lax.fori_loop(0, NCH, cnt_chunk, _i32(0))
    put_row(spcnt, t, mycnt)
    plsc.subcore_barrier()
    pltpu.sync_copy(spcnt, cntv)
    off = _i32(0)
    tot = _i32(0)
    for r in range(NT):
      tr = _scalar(cntv[r])
      off = off + jnp.where(r < t, tr, 0).astype(_i32)
      tot = tot + tr
    gbase = s * K_KEEP + off

    def flush(src_ref, lo_row, count, ngroups):
      def fl(j, _):
        for o in range(8):
          p = j * 128 + o * 16 + iota
          rowv = jnp.where(p < count, lo_row + p, DUMP + o * 16 + iota)
          idx2d[j, pl.ds(o * 16, 16)] = rowv.astype(_i32)
        if src_ref is aos:
          pltpu.async_copy(aos.at[pl.ds(j * 128, 128)],
                           out.at[idx2d.at[j]], sem).wait()
        else:
          pltpu.async_copy(src_ref.at[...],
                           out.at[idx2d.at[j]], sem).wait()
        return 0
      lax.fori_loop(0, ngroups, fl, 0)

    def out_chunk(i, done):
      pltpu.sync_copy(rdr.at[s, pl.ds(shard + i * CH, CH)], buf)
      def vl(v, crun):
        x = buf[pl.ds(v * 16, 16)]
        mask = x > threshv
        mi = mask.astype(_i32)
        pos = jnp.maximum(crun + plsc.cumsum(mi) - 1, 0)
        g = shard + i * CH + v * 16 + iota
        z = lax.shift_right_logical(g, 16)
        rem = jnp.bitwise_and(g, 65535)
        y = lax.shift_right_logical(rem, 8)
        xq = jnp.bitwise_and(rem, 255)
        xc = xq.astype(_f32) * _f32(0.3125)
        yc = y.astype(_f32) * _f32(0.3125) - _f32(40.0)
        zc = z.astype(_f32) * _f32(0.25) - _f32(2.0)
        pw = x / _f32(1.0e13)
        sv = jnp.broadcast_to(s.astype(_i32), (16,))
        cols = [lax.bitcast_convert_type(xc, _i32),
                lax.bitcast_convert_type(yc, _i32),
                lax.bitcast_convert_type(zc, _i32),
                lax.bitcast_convert_type(pw, _i32),
                sv, z, y, xq]
        for f in range(8):
          plsc.store_scatter(aos, [pos, jnp.broadcast_to(_i32(f), (16,))],
                             cols[f], mask=mask)
        return crun + jnp.sum(mi)
      crun = lax.fori_loop(0, CH // 16, vl, _i32(0))
      flush(aos, gbase + done, crun, (crun + 127) // 128)
      return done + crun
    lax.fori_loop(0, NCH, out_chunk, _i32(0))

    padcnt = K_KEEP - tot
    share = (padcnt + NT - 1) // NT
    mystart = t * share
    myp = jnp.clip(padcnt - mystart, 0, share)

    @pl.when(myp > 0)
    def _():
      pltpu.sync_copy(rdr.at[s, pl.ds(0, 16)], buf.at[pl.ds(0, 16)])
      xv = buf[pl.ds(0, 16)] / _f32(1.0e13)
      p0 = jnp.sum(jnp.where(iota == 0, xv, _f32(0.0)))
      p0b = lax.bitcast_convert_type(jnp.broadcast_to(p0, (16,)), _i32)
      sv = jnp.broadcast_to(s.astype(_i32), (16,))
      pv = jnp.where(iota == 1, B_Y, jnp.where(iota == 2, B_Z,
           jnp.where(iota == 3, p0b, jnp.where(iota == 4, sv, 0)))).astype(_i32)
      def pf(gi, _):
        rows = jnp.broadcast_to(gi, (16,))
        plsc.store_scatter(padbuf, [rows, iota], pv)
        return 0
      lax.fori_loop(0, 128, pf, 0)
      flush(padbuf, s * K_KEEP + tot + mystart, myp, (myp + 127) // 128)

    plsc.subcore_barrier()
    return 0

  lax.fori_loop(0, 2, sample_body, 0)


def _run(rdr_flat):
  mesh = plsc.VectorSubcoreMesh(core_axis_name="c", subcore_axis_name="s",
                                num_cores=2, num_subcores=NT)
  kern = pl.kernel(
      _kernel_body,
      out_type=jax.ShapeDtypeStruct((OUT_PAD, OW), jnp.int32),
      mesh=mesh,
      compiler_params=pltpu.CompilerParams(
          needs_layout_passes=False, use_tc_tiling_on_sc=False),
      scratch_types=[
          pltpu.VMEM((CH,), _f32),
          pltpu.VMEM((NBINS // 16, 16), _i32),
          pltpu.VMEM((CHR, 16), _i32),
          pltpu.VMEM((CHR, 16), _i32),
          pltpu.VMEM((CH, OW), _i32),
          pltpu.VMEM((16, 128), _i32),
          pltpu.VMEM((128, OW), _i32),
          pltpu.VMEM((NT, 16), _i32),
          pltpu.VMEM((8, 16), _i32),
          pltpu.VMEM((NT, 16), _i32),
          pltpu.VMEM((NT, 16), _i32),
          pltpu.VMEM((16,), _i32),
          pltpu.VMEM_SHARED((NBINS // 16, 16), _i32),
          pltpu.VMEM_SHARED((NT, 16), _i32),
          pltpu.VMEM_SHARED((8, 16), _i32),
          pltpu.VMEM_SHARED((NT, 16), _i32),
          pltpu.VMEM_SHARED((NT, 16), _i32),
          pltpu.SemaphoreType.DMA,
      ],
  )
  return kern(rdr_flat)


@jax.jit
def kernel(rdr_cube, batch_size):
  rdr_flat = rdr_cube.reshape(B, N)
  out = _run(rdr_flat)
  rows = out[:R_OUT, :8]
  feats = lax.bitcast_convert_type(rows[:, 0:4], jnp.float32)
  inds = rows[:, 4:8]
  return feats, inds


# speedup vs baseline: 19.5594x; 1.0713x over previous
import jax
import jax.numpy as jnp
from jax import lax
from jax.experimental import pallas as pl
from jax.experimental.pallas import tpu as pltpu
from jax.experimental.pallas import tpu_sc as plsc

B = 4
ZD, YD, XD = 32, 256, 256
N = ZD * YD * XD
NT = 16
SH = N // NT
CH = 2048
NCH = SH // CH
K0 = 1468005
K_KEEP = N - K0 - 1
R_OUT = B * K_KEEP
DUMP = R_OUT
OUT_PAD = R_OUT + 128
OW = 16
NBINS = 65536
CHB = NBINS // NT
CHR = CHB // 16
HW = 0.625
INT_MIN = -2147483648

_i32 = jnp.int32
_f32 = jnp.float32


def _bits(v):
  import struct
  return int.from_bytes(struct.pack('<f', v), 'little', signed=True)


B_Y = _bits(-40.0)
B_Z = _bits(-2.0)


def _monotonic(x):
  xi = lax.bitcast_convert_type(x, _i32)
  return jnp.where(xi < 0, jnp.bitwise_not(xi),
                   jnp.bitwise_or(xi, _i32(INT_MIN)))


def _scalar(vec):
  return jnp.max(vec)


def _kernel_body(rdr, out, buf, buf2, hist, acc, zb, aos, idx2d, padbuf,
                 totv, resv, minv, cntv, stage, sphist, sptot, spres,
                 spmin, spcnt, sem, semA, semB):
  c = lax.axis_index("c")
  t = lax.axis_index("s")
  iota = lax.iota(_i32, 16)

  def zero_rows(ref, nrows):
    def zh(i, _):
      ref[i] = jnp.zeros((16,), _i32)
      return 0
    lax.fori_loop(0, nrows, zh, 0)

  def put_row(sp, row, scalar_val):
    stage[...] = jnp.broadcast_to(scalar_val.astype(_i32), (16,))
    pltpu.sync_copy(stage, sp.at[row])

  def merge_hist_to_spmem():
    pltpu.sync_copy(zb, sphist.at[pl.ds(t * CHR, CHR)])
    plsc.subcore_barrier()
    def mg(j, _):
      for o in range(8):
        idx2d[j, pl.ds(o * 16, 16)] = j * 128 + o * 16 + iota
      pltpu.sync_copy(hist.at[pl.ds(j * 128, 128)],
                      sphist.at[idx2d.at[j]], add=True)
      return 0
    lax.fori_loop(0, NBINS // 16 // 128, mg, 0)
    plsc.subcore_barrier()

  def reduce_and_cumsum():
    pltpu.sync_copy(sphist.at[pl.ds(t * CHR, CHR)], acc)
    def csum(i, run):
      v = acc[i]
      acc[i] = plsc.cumsum(v) + run
      return run + jnp.sum(v)
    total = lax.fori_loop(0, CHR, csum, _i32(0))
    put_row(sptot, t, total)
    plsc.subcore_barrier()
    pltpu.sync_copy(sptot, totv)
    base = _i32(0)
    for r in range(NT):
      tr = _scalar(totv[r])
      base = base + jnp.where(r < t, tr, 0).astype(_i32)
    return base, total

  def rank_search(base, total, k, row_bin, row_cnt):
    def lp(i, cnt):
      v = acc[i] + base
      return cnt + jnp.sum(jnp.where(v <= k, 1, 0).astype(_i32))
    L = lax.fori_loop(0, CHR, lp, _i32(0))
    claim = jnp.logical_and(k >= base, k < base + total)

    @pl.when(claim)
    def _():
      lm = jnp.maximum(L - 1, 0)
      gv = plsc.load_gather(
          acc, [jnp.broadcast_to(lax.shift_right_logical(lm, 4), (16,)),
                jnp.broadcast_to(jnp.bitwise_and(lm, 15), (16,))])
      below = base + jnp.where(L > 0, _scalar(gv), 0).astype(_i32)
      put_row(spres, row_bin, t * CHB + L)
      if row_cnt is not None:
        put_row(spres, row_cnt, below)

  zero_rows(zb, CHR)

  def stream_pass(s, shard, body, init):
    pltpu.async_copy(rdr.at[s, pl.ds(shard, CH)], buf, semA)
    def lp(k, carry):
      i0 = 2 * k
      i1 = i0 + 1
      pltpu.make_async_copy(rdr.at[s, pl.ds(shard + i0 * CH, CH)],
                            buf, semA).wait()
      pltpu.async_copy(rdr.at[s, pl.ds(shard + i1 * CH, CH)], buf2, semB)
      carry = body(buf, i0, carry)
      pltpu.make_async_copy(rdr.at[s, pl.ds(shard + i1 * CH, CH)],
                            buf2, semB).wait()
      @pl.when(i1 + 1 < NCH)
      def _():
        pltpu.async_copy(rdr.at[s, pl.ds(shard + (i1 + 1) * CH, CH)],
                         buf, semA)
      return body(buf2, i1, carry)
    return lax.fori_loop(0, NCH // 2, lp, init)

  def sample_body(sl, _):
    s = c * 2 + sl
    shard = t * SH

    zero_rows(hist, NBINS // 16)
    def p1_chunk(bf, i, __):
      def vl(v, ___):
        x = bf[pl.ds(v * 16, 16)]
        hi = lax.shift_right_logical(_monotonic(x), 16)
        cnt, last = plsc.scan_count(hi)
        plsc.addupdate_scatter(
            hist, [lax.shift_right_logical(hi, 4), jnp.bitwise_and(hi, 15)],
            cnt, mask=last)
        return 0
      lax.fori_loop(0, CH // 16, vl, 0)
      return 0
    stream_pass(s, shard, p1_chunk, 0)
    merge_hist_to_spmem()

    base, total = reduce_and_cumsum()
    rank_search(base, total, _i32(K0), 0, 1)
    rank_search(base, total, _i32(K0 + 1), 2, 3)
    plsc.subcore_barrier()
    pltpu.sync_copy(spres, resv)
    b0 = _scalar(resv[0])
    cb0 = _scalar(resv[1])
    b1 = _scalar(resv[2])
    cb1 = _scalar(resv[3])

    zero_rows(hist, NBINS // 16)
    def p2_chunk(bf, i, mlow):
      def vl(v, ml):
        x = bf[pl.ds(v * 16, 16)]
        m = _monotonic(x)
        hi = lax.shift_right_logical(m, 16)
        lo = jnp.bitwise_and(m, 65535)
        cnt, last = plsc.scan_count(lo, mask=hi == b0)
        plsc.addupdate_scatter(
            hist, [lax.shift_right_logical(lo, 4), jnp.bitwise_and(lo, 15)],
            cnt, mask=last)
        lv = jnp.where(hi == b1, lo, 70000)
        return jnp.minimum(ml, jnp.min(lv)).astype(_i32)
      return lax.fori_loop(0, CH // 16, vl, mlow)
    minlow = stream_pass(s, shard, p2_chunk, _i32(70000))
    put_row(spmin, t, minlow)
    merge_hist_to_spmem()

    base2, total2 = reduce_and_cumsum()
    rank_search(base2, total2, K0 - cb0, 4, None)
    rank_search(base2, total2, K0 + 1 - cb1, 5, None)
    plsc.subcore_barrier()
    pltpu.sync_copy(spres, resv)
    pltpu.sync_copy(spmin, minv)
    l0 = _scalar(resv[4])
    l1 = _scalar(resv[5])
    mlg = _i32(70000)
    for r in range(NT):
      mlg = jnp.minimum(mlg, _scalar(minv[r]))
    l1s = jnp.where(b1 == b0, l1, mlg).astype(_i32)
    v0m = jnp.bitwise_or(lax.shift_left(b0, 16), l0)
    v1m = jnp.bitwise_or(lax.shift_left(b1, 16), l1s)

    def m2f(mv):
      vec = jnp.broadcast_to(mv, (16,))
      bits = jnp.where(vec < 0, jnp.bitwise_and(vec, 2147483647),
                       jnp.bitwise_not(vec))
      return lax.bitcast_convert_type(bits.astype(_i32), _f32)
    v0f = m2f(v0m)
    v1f = m2f(v1m)
    threshv = v0f + _f32(HW) * (v1f - v0f)

    def cnt_chunk(bf, i, cc):
      def vl(v, c2):
        x = bf[pl.ds(v * 16, 16)]
        return c2 + jnp.sum(jnp.where(x > threshv, 1, 0).astype(_i32))
      return lax.fori_loop(0, CH // 16, vl, cc)
    mycnt = stream_pass(s, shard, cnt_chunk, _i32(0))
    put_row(spcnt, t, mycnt)
    plsc.subcore_barrier()
    pltpu.sync_copy(spcnt, cntv)
    off = _i32(0)
    tot = _i32(0)
    for r in range(NT):
      tr = _scalar(cntv[r])
      off = off + jnp.where(r < t, tr, 0).astype(_i32)
      tot = tot + tr
    gbase = s * K_KEEP + off

    def flush(src_ref, lo_row, count, ngroups):
      def fl(j, _):
        for o in range(8):
          p = j * 128 + o * 16 + iota
          rowv = jnp.where(p < count, lo_row + p, DUMP + o * 16 + iota)
          idx2d[j, pl.ds(o * 16, 16)] = rowv.astype(_i32)
        if src_ref is aos:
          pltpu.async_copy(aos.at[pl.ds(j * 128, 128)],
                           out.at[idx2d.at[j]], sem).wait()
        else:
          pltpu.async_copy(src_ref.at[...],
                           out.at[idx2d.at[j]], sem).wait()
        return 0
      lax.fori_loop(0, ngroups, fl, 0)

    def out_chunk(bf, i, done):
      def vl(v, crun):
        x = bf[pl.ds(v * 16, 16)]
        mask = x > threshv
        mi = mask.astype(_i32)
        pos = jnp.maximum(crun + plsc.cumsum(mi) - 1, 0)
        g = shard + i * CH + v * 16 + iota
        z = lax.shift_right_logical(g, 16)
        rem = jnp.bitwise_and(g, 65535)
        y = lax.shift_right_logical(rem, 8)
        xq = jnp.bitwise_and(rem, 255)
        xc = xq.astype(_f32) * _f32(0.3125)
        yc = y.astype(_f32) * _f32(0.3125) - _f32(40.0)
        zc = z.astype(_f32) * _f32(0.25) - _f32(2.0)
        pw = x / _f32(1.0e13)
        sv = jnp.broadcast_to(s.astype(_i32), (16,))
        cols = [lax.bitcast_convert_type(xc, _i32),
                lax.bitcast_convert_type(yc, _i32),
                lax.bitcast_convert_type(zc, _i32),
                lax.bitcast_convert_type(pw, _i32),
                sv, z, y, xq]
        for f in range(8):
          plsc.store_scatter(aos, [pos, jnp.broadcast_to(_i32(f), (16,))],
                             cols[f], mask=mask)
        return crun + jnp.sum(mi)
      crun = lax.fori_loop(0, CH // 16, vl, _i32(0))
      flush(aos, gbase + done, crun, (crun + 127) // 128)
      return done + crun
    stream_pass(s, shard, out_chunk, _i32(0))

    padcnt = K_KEEP - tot
    share = (padcnt + NT - 1) // NT
    mystart = t * share
    myp = jnp.clip(padcnt - mystart, 0, share)

    @pl.when(myp > 0)
    def _():
      pltpu.sync_copy(rdr.at[s, pl.ds(0, 16)], buf.at[pl.ds(0, 16)])
      xv = buf[pl.ds(0, 16)] / _f32(1.0e13)
      p0 = jnp.sum(jnp.where(iota == 0, xv, _f32(0.0)))
      p0b = lax.bitcast_convert_type(jnp.broadcast_to(p0, (16,)), _i32)
      sv = jnp.broadcast_to(s.astype(_i32), (16,))
      pv = jnp.where(iota == 1, B_Y, jnp.where(iota == 2, B_Z,
           jnp.where(iota == 3, p0b, jnp.where(iota == 4, sv, 0)))).astype(_i32)
      def pf(gi, _):
        rows = jnp.broadcast_to(gi, (16,))
        plsc.store_scatter(padbuf, [rows, iota], pv)
        return 0
      lax.fori_loop(0, 128, pf, 0)
      flush(padbuf, s * K_KEEP + tot + mystart, myp, (myp + 127) // 128)

    plsc.subcore_barrier()
    return 0

  lax.fori_loop(0, 2, sample_body, 0)


def _run(rdr_flat):
  mesh = plsc.VectorSubcoreMesh(core_axis_name="c", subcore_axis_name="s",
                                num_cores=2, num_subcores=NT)
  kern = pl.kernel(
      _kernel_body,
      out_type=jax.ShapeDtypeStruct((OUT_PAD, OW), jnp.int32),
      mesh=mesh,
      compiler_params=pltpu.CompilerParams(
          needs_layout_passes=False, use_tc_tiling_on_sc=False),
      scratch_types=[
          pltpu.VMEM((CH,), _f32),
          pltpu.VMEM((CH,), _f32),
          pltpu.VMEM((NBINS // 16, 16), _i32),
          pltpu.VMEM((CHR, 16), _i32),
          pltpu.VMEM((CHR, 16), _i32),
          pltpu.VMEM((CH, OW), _i32),
          pltpu.VMEM((16, 128), _i32),
          pltpu.VMEM((128, OW), _i32),
          pltpu.VMEM((NT, 16), _i32),
          pltpu.VMEM((8, 16), _i32),
          pltpu.VMEM((NT, 16), _i32),
          pltpu.VMEM((NT, 16), _i32),
          pltpu.VMEM((16,), _i32),
          pltpu.VMEM_SHARED((NBINS // 16, 16), _i32),
          pltpu.VMEM_SHARED((NT, 16), _i32),
          pltpu.VMEM_SHARED((8, 16), _i32),
          pltpu.VMEM_SHARED((NT, 16), _i32),
          pltpu.VMEM_SHARED((NT, 16), _i32),
          pltpu.SemaphoreType.DMA,
          pltpu.SemaphoreType.DMA,
          pltpu.SemaphoreType.DMA,
      ],
  )
  return kern(rdr_flat)


@jax.jit
def kernel(rdr_cube, batch_size):
  rdr_flat = rdr_cube.reshape(B, N)
  out = _run(rdr_flat)
  rows = out[:R_OUT, :8]
  feats = lax.bitcast_convert_type(rows[:, 0:4], jnp.float32)
  inds = rows[:, 4:8]
  return feats, inds
